# Initial kernel scaffold; baseline (speedup 1.0000x reference)
import jax
import jax.numpy as jnp
from jax.experimental import pallas as pl


def kernel(x_s, edge_index_s, x_f, edge_index_f, idx, params):
    raise NotImplementedError("write your pallas kernel here")


# speedup vs baseline: 5.2586x; 5.2586x over previous
import functools

import jax
import jax.numpy as jnp
from jax import lax
from jax.experimental import pallas as pl
from jax.experimental.pallas import tpu as pltpu
from jax.experimental.pallas import tpu_sc as plsc

N = 10000
E = 320000
D = 128
HEADS = 8
HD = 16

NC = 2
NS = 16
NW = NC * NS
CHUNK = 128
NPAD = 10240
EPT = 10240
EPAD = NW * EPT
NCHUNK = EPT // CHUNK
RPT_SC = NPAD // NS


def _mesh():
    return plsc.VectorSubcoreMesh(core_axis_name="c", subcore_axis_name="s",
                                  num_cores=NC, num_subcores=NS)


DW = D // NC
EPT2 = EPAD // NS
NCHUNK2 = EPT2 // CHUNK


@functools.cache
def _build_sc_conv():
    @functools.partial(
        pl.kernel,
        out_type=jax.ShapeDtypeStruct((NC, NPAD, DW), jnp.float32),
        mesh=_mesh(),
        compiler_params=pltpu.CompilerParams(use_tc_tiling_on_sc=False),
        scratch_types=[
            pltpu.VMEM((NCHUNK2, CHUNK), jnp.int32),
            pltpu.VMEM((NCHUNK2, CHUNK), jnp.int32),
            pltpu.VMEM((2, CHUNK, DW), jnp.float32),
            pltpu.VMEM_SHARED((NPAD, DW), jnp.float32),
            pltpu.SemaphoreType.DMA,
            pltpu.SemaphoreType.DMA,
        ],
    )
    def sc_conv(table_h, src_h, dst_h, zeros_h, out_h, sidx, didx, rows, acc, sem0, sem1):
        cid = lax.axis_index("c")
        sid = lax.axis_index("s")
        pltpu.sync_copy(zeros_h, acc.at[pl.ds(sid * RPT_SC, RPT_SC)])
        pltpu.sync_copy(src_h.at[pl.ds(sid * NCHUNK2, NCHUNK2)], sidx)
        pltpu.sync_copy(dst_h.at[pl.ds(sid * NCHUNK2, NCHUNK2)], didx)
        plsc.subcore_barrier()

        tab = table_h.at[cid]
        pltpu.async_copy(tab.at[sidx.at[0]], rows.at[0], sem0)

        def body(t, carry):
            k0 = 2 * t
            pltpu.async_copy(tab.at[sidx.at[k0 + 1]], rows.at[1], sem1)
            pltpu.make_async_copy(tab.at[sidx.at[k0]], rows.at[0], sem0).wait()
            pltpu.sync_copy(rows.at[0], acc.at[didx.at[k0]], add=True)

            @pl.when(t + 1 < NCHUNK2 // 2)
            def _():
                pltpu.async_copy(tab.at[sidx.at[k0 + 2]], rows.at[0], sem0)

            pltpu.make_async_copy(tab.at[sidx.at[k0 + 1]], rows.at[1], sem1).wait()
            pltpu.sync_copy(rows.at[1], acc.at[didx.at[k0 + 1]], add=True)
            return carry

        lax.fori_loop(0, NCHUNK2 // 2, body, 0)
        plsc.subcore_barrier()
        pltpu.sync_copy(acc.at[pl.ds(sid * RPT_SC, RPT_SC)],
                        out_h.at[cid, pl.ds(sid * RPT_SC, RPT_SC)])

    return sc_conv


def _sc_conv(table, src2d, dst2d, zeros_sc):
    tsplit = table.reshape(NPAD, NC, DW).transpose(1, 0, 2)
    out = _build_sc_conv()(tsplit, src2d, dst2d, zeros_sc)
    return out.transpose(1, 0, 2).reshape(NPAD, D)


DEGW = 16


@functools.cache
def _build_sc_degree():
    @functools.partial(
        pl.kernel,
        out_type=jax.ShapeDtypeStruct((NC, NPAD, DEGW), jnp.float32),
        mesh=_mesh(),
        compiler_params=pltpu.CompilerParams(use_tc_tiling_on_sc=False),
        scratch_types=[
            pltpu.VMEM((NCHUNK, CHUNK), jnp.int32),
            pltpu.VMEM((CHUNK, DEGW), jnp.float32),
            pltpu.VMEM_SHARED((NPAD, DEGW), jnp.float32),
        ],
    )
    def sc_degree(dst_h, ones_h, zeros_h, out_h, didx, ones_v, acc):
        cid = lax.axis_index("c")
        sid = lax.axis_index("s")
        g = cid * NS + sid
        pltpu.sync_copy(dst_h.at[pl.ds(g * NCHUNK, NCHUNK)], didx)
        pltpu.sync_copy(ones_h, ones_v)
        pltpu.sync_copy(zeros_h, acc.at[pl.ds(sid * RPT_SC, RPT_SC)])
        plsc.subcore_barrier()

        def body(k, c):
            pltpu.sync_copy(ones_v, acc.at[didx.at[k]], add=True)
            return c

        lax.fori_loop(0, NCHUNK, body, 0)
        plsc.subcore_barrier()
        pltpu.sync_copy(acc.at[pl.ds(sid * RPT_SC, RPT_SC)],
                        out_h.at[cid, pl.ds(sid * RPT_SC, RPT_SC)])

    return sc_degree


def _sc_degree(dst2d):
    ones_rows = jnp.ones((CHUNK, DEGW), jnp.float32)
    zeros_rows = jnp.zeros((RPT_SC, DEGW), jnp.float32)
    return _build_sc_degree()(dst2d, ones_rows, zeros_rows)[:, :, 0]


@functools.cache
def _build_sc_take():
    @functools.partial(
        pl.kernel,
        out_type=jax.ShapeDtypeStruct((8192, D), jnp.float32),
        mesh=_mesh(),
        compiler_params=pltpu.CompilerParams(use_tc_tiling_on_sc=False),
        scratch_types=[
            pltpu.VMEM((2, CHUNK), jnp.int32),
            pltpu.VMEM((2 * CHUNK, D), jnp.float32),
            pltpu.SemaphoreType.DMA,
        ],
    )
    def sc_take(table_h, idx_h, out_h, gidx, rows, sem):
        cid = lax.axis_index("c")
        sid = lax.axis_index("s")
        g = cid * NS + sid
        pltpu.sync_copy(idx_h.at[pl.ds(g * 2, 2)], gidx)
        pltpu.async_copy(table_h.at[gidx.at[0]], rows.at[pl.ds(0, CHUNK)], sem).wait()
        pltpu.async_copy(table_h.at[gidx.at[1]], rows.at[pl.ds(CHUNK, CHUNK)], sem).wait()
        pltpu.sync_copy(rows, out_h.at[pl.ds(g * 2 * CHUNK, 2 * CHUNK)])

    return sc_take


def _sc_take(table, idx2d):
    return _build_sc_take()(table, idx2d)


def _pad_rows(a, m):
    r = a.shape[0] % m
    if r == 0:
        return a
    return jnp.concatenate([a, jnp.zeros((m - r,) + a.shape[1:], a.dtype)], axis=0)


def _mm(a, w, block_m=512):
    m, k = a.shape
    ko = w.shape[1]
    kop = ((ko + 127) // 128) * 128
    if kop != ko:
        w = jnp.concatenate([w, jnp.zeros((k, kop - ko), w.dtype)], axis=1)
    ap = _pad_rows(a, block_m)
    grid = ap.shape[0] // block_m

    def body(ar, wr, orr):
        orr[...] = jnp.dot(ar[...], wr[...], preferred_element_type=jnp.float32)

    out = pl.pallas_call(
        body,
        grid=(grid,),
        in_specs=[
            pl.BlockSpec((block_m, k), lambda i: (i, 0)),
            pl.BlockSpec((k, kop), lambda i: (0, 0)),
        ],
        out_specs=pl.BlockSpec((block_m, kop), lambda i: (i, 0)),
        out_shape=jax.ShapeDtypeStruct((ap.shape[0], kop), jnp.float32),
    )(ap, w)
    if kop != ko:
        out = out[:, :ko]
    if out.shape[0] != m:
        out = out[:m]
    return out


def _colstats(a, block_m=1024):
    ap = _pad_rows(a, block_m)
    grid = ap.shape[0] // block_m

    def body(ar, orr):
        @pl.when(pl.program_id(0) == 0)
        def _():
            orr[...] = jnp.zeros_like(orr)

        x = ar[...]
        s = jnp.sum(x, axis=0, keepdims=True)
        sq = jnp.sum(x * x, axis=0, keepdims=True)
        orr[...] += jnp.concatenate([s, sq, jnp.zeros((6, D), jnp.float32)], axis=0)

    return pl.pallas_call(
        body,
        grid=(grid,),
        in_specs=[pl.BlockSpec((block_m, D), lambda i: (i, 0))],
        out_specs=pl.BlockSpec((8, D), lambda i: (0, 0)),
        out_shape=jax.ShapeDtypeStruct((8, D), jnp.float32),
    )(ap)


def _celu2(x):
    return jnp.where(x > 0, x, 2.0 * (jnp.exp(0.5 * x) - 1.0))


def _dfam_kernel(qkv, epad, epadt, aggw, agg2v, pvec, block_m=512):
    grid = NPAD // block_m

    def body(q0r, q1r, q2r, k0r, k1r, k2r, v0r, v1r, v2r, er, etr, awr, a2r, pvr, orr):
        scale = float(HD) ** -0.5
        q = [q0r[...], q1r[...], q2r[...]]
        kk = [k0r[...], k1r[...], k2r[...]]
        v = [v0r[...], v1r[...], v2r[...]]
        e = er[...]
        et = etr[...]
        pv = pvr[...]
        a1b = pv[0:1, :]
        alpha = pv[1:2, :]
        dw = pv[2:3, :]
        db = pv[3:4, :]
        a2 = a2r[...]

        outs = []
        ws = []
        for i in range(3):
            s = [jnp.dot(q[i] * kk[j], e, preferred_element_type=jnp.float32) * scale
                 for j in range(3)]
            mx = jnp.maximum(jnp.maximum(s[0], s[1]), s[2])
            ex = [jnp.exp(sj - mx) for sj in s]
            z = ex[0] + ex[1] + ex[2]
            o_i = jnp.zeros_like(q[0])
            for j in range(3):
                p_l = jnp.dot(ex[j] / z, et, preferred_element_type=jnp.float32)
                o_i = o_i + p_l * v[j]
            o_i = _celu2(o_i)
            d_i = jnp.tanh(alpha * (jnp.dot(o_i, awr[...], preferred_element_type=jnp.float32) + a1b)) * dw + db
            w_i = jnp.sum(d_i * a2, axis=1, keepdims=True)
            outs.append(o_i)
            ws.append(w_i)
        wm = jnp.maximum(jnp.maximum(ws[0], ws[1]), ws[2])
        ew = [jnp.exp(wi - wm) for wi in ws]
        zw = ew[0] + ew[1] + ew[2]
        orr[...] = (ew[0] * outs[0] + ew[1] * outs[1] + ew[2] * outs[2]) / zw

    def bspec(c0):
        return pl.BlockSpec((block_m, D), lambda i: (i, c0))

    def cspec(r, c):
        return pl.BlockSpec((r, c), lambda i: (0, 0))

    q0, q1, q2 = qkv[0:NPAD], qkv[NPAD:2 * NPAD], qkv[2 * NPAD:3 * NPAD]

    return pl.pallas_call(
        body,
        grid=(grid,),
        in_specs=[bspec(0), bspec(0), bspec(0),
                  bspec(1), bspec(1), bspec(1),
                  bspec(2), bspec(2), bspec(2),
                  cspec(D, D), cspec(D, D), cspec(D, D),
                  cspec(1, D), cspec(8, D)],
        out_specs=pl.BlockSpec((block_m, D), lambda i: (i, 0)),
        out_shape=jax.ShapeDtypeStruct((NPAD, D), jnp.float32),
    )(q0, q1, q2, q0, q1, q2, q0, q1, q2, epad, epadt, aggw, agg2v, pvec)


def _dtf(x, p):
    return jnp.tanh(p['alpha'] * x) * p['weight'] + p['bias']


def _dgct_gate(colsq, p, eps=1e-05):
    emb = jnp.power(colsq + eps, 0.5) * p['alpha']
    gamma = _dtf(emb, p['dyt_gamma'])
    norm = gamma / jnp.power(jnp.mean(emb * emb, axis=1, keepdims=True) + eps, 0.5)
    beta = _dtf(emb, p['dyt_beta'])
    return 1.0 + jax.nn.celu(emb * norm + beta, alpha=2.0)


def _prep_edges(ei):
    pad = EPAD - E
    src = jnp.concatenate([ei[0], jnp.full((pad,), N, jnp.int32)]).reshape(EPAD // CHUNK, CHUNK)
    dst = jnp.concatenate([ei[1], jnp.full((pad,), N, jnp.int32)]).reshape(EPAD // CHUNK, CHUNK)
    return src, dst


def _conv(xw_pad, dis_pad, src2d, dst2d, b, zeros_sc):
    xwp = dis_pad * xw_pad
    s = _sc_conv(xwp, src2d, dst2d, zeros_sc)
    return dis_pad * (s + xwp) + b


def _encoder(x_pad, colsq, src2d, dst2d, dis_pad, p, zeros_sc):
    g1 = _dgct_gate(colsq, p['gct1'])
    g2 = _dgct_gate(colsq, p['gct2'])
    x0 = x_pad * g1
    x00 = x_pad * g2
    xw = _mm(jnp.concatenate([x0, x00], axis=0), p['gc1a_W'])
    ca1 = _conv(xw[:NPAD], dis_pad, src2d, dst2d, p['gc1a_b'], zeros_sc)
    cb1 = _conv(xw[NPAD:], dis_pad, src2d, dst2d, p['gc1a_b'], zeros_sc)
    x1a = _dtf(jax.nn.celu(ca1, alpha=2.0), p['norm1'])
    x1b = _dtf(cb1, p['norm1'])
    xw2 = _mm(jnp.concatenate([x1a, x1b], axis=0), p['gc2_W'])
    ca2 = _conv(xw2[:NPAD], dis_pad, src2d, dst2d, p['gc2_b'], zeros_sc)
    cb2 = _conv(xw2[NPAD:], dis_pad, src2d, dst2d, p['gc2_b'], zeros_sc)
    x1aa = _dtf(jax.nn.celu(ca2, alpha=2.0), p['norm2'])
    x1bb = jax.nn.sigmoid(cb2)
    return x1aa * x1bb


def kernel(x_s, edge_index_s, x_f, edge_index_f, idx, params):
    p = params
    zeros_sc = jnp.zeros((RPT_SC, DW), jnp.float32)

    src_s, dst_s = _prep_edges(edge_index_s)
    src_f, dst_f = _prep_edges(edge_index_f)

    degp_s = _sc_degree(dst_s)
    degp_f = _sc_degree(dst_f)
    deg_s = degp_s[0] + degp_s[1] + 1.0
    deg_f = degp_f[0] + degp_f[1] + 1.0
    dis_s = lax.rsqrt(deg_s)
    dis_f = lax.rsqrt(deg_f)
    rowmask = (jnp.arange(NPAD) < N).astype(jnp.float32)
    dis_s_pad = (dis_s * rowmask)[:, None]
    dis_f_pad = (dis_f * rowmask)[:, None]

    xs_pad = _pad_rows(x_s, NPAD)
    xf_pad = _pad_rows(x_f, NPAD)
    colsq_xs = _colstats(x_s)[1:2, :]
    colsq_xf = _colstats(x_f)[1:2, :]

    h1 = _encoder(xs_pad, colsq_xs, src_s, dst_s, dis_s_pad, p['enc1'], zeros_sc)
    h2 = _encoder(xs_pad, colsq_xs, src_f, dst_f, dis_f_pad, p['enc2'], zeros_sc)
    h3 = _encoder(xf_pad, colsq_xf, src_s, dst_s, dis_s_pad, p['enc1'], zeros_sc)
    h4 = _encoder(xf_pad, colsq_xf, src_f, dst_f, dis_f_pad, p['enc2'], zeros_sc)
    h5 = _encoder(xs_pad, colsq_xs, src_s, dst_s, dis_s_pad, p['enc3'], zeros_sc)
    h6 = _encoder(xs_pad, colsq_xs, src_f, dst_f, dis_f_pad, p['enc3'], zeros_sc)

    hloc = jax.nn.celu(
        _mm(jnp.concatenate([h1, h2, h3, h4], axis=0), p['local_W']) + p['local_b'],
        alpha=2.0)
    h1l, h2l, h3l, h4l = (hloc[i * NPAD:(i + 1) * NPAD] for i in range(4))
    h_com = (h5 + h6) * 0.5

    m1 = _colstats(h1l[:N])[0] / N
    m2 = _colstats(h2l[:N])[0] / N
    cc = jax.nn.sigmoid(_mm(jnp.stack([m1, m2], axis=0), p['global_W']) + p['global_b'])
    c1, c2 = cc[0], cc[1]

    wc = _mm(p['disc_W'], jnp.stack([c1, c2], axis=1))
    hcat = jnp.concatenate([h1l, h3l, h2l, h4l], axis=0)
    sc_all = _mm(hcat, wc)
    sc1 = sc_all[0:N, 0] + p['disc_b']
    sc3 = sc_all[NPAD:NPAD + N, 0] + p['disc_b']
    sc2 = sc_all[2 * NPAD:2 * NPAD + N, 1] + p['disc_b']
    sc4 = sc_all[3 * NPAD:3 * NPAD + N, 1] + p['disc_b']
    out = jax.nn.celu(jnp.concatenate([sc1, sc2, sc3, sc4]), alpha=2.0)

    ap = p['attn']
    wqkv = jnp.concatenate([ap['q_W'], ap['k_W'], ap['v_W']], axis=1)
    bqkv = jnp.concatenate([ap['q_b'], ap['k_b'], ap['v_b']])
    emb_rows = jnp.concatenate([h1l, h2l, h_com], axis=0)
    qkv = _mm(emb_rows, wqkv) + bqkv

    eye8 = jnp.eye(HEADS, dtype=jnp.float32)
    epad_small = jnp.repeat(eye8, HD, axis=0)
    epad = jnp.concatenate([epad_small, jnp.zeros((D, D - HEADS), jnp.float32)], axis=1)
    epadt = epad.T
    pvec = jnp.zeros((8, D), jnp.float32)
    pvec = pvec.at[0].set(ap['agg1_b'])
    pvec = pvec.at[1].set(jnp.full((D,), ap['agg_dtf']['alpha'][0]))
    pvec = pvec.at[2].set(ap['agg_dtf']['weight'])
    pvec = pvec.at[3].set(ap['agg_dtf']['bias'])
    agg2v = ap['agg2_W'].reshape(1, D)
    agg = _dfam_kernel(qkv, epad, epadt, ap['agg1_W'], agg2v, pvec)

    idx_cat = jnp.concatenate([idx[0], idx[1] + 386]).reshape(NW * 2, CHUNK)
    e12 = _sc_take(agg, idx_cat)
    e1, e2 = e12[:4096], e12[4096:]
    feature = jnp.concatenate([e1 + e2, e1 * e2, e1, e2], axis=1)
    log1 = jax.nn.celu(_mm(feature, p['dec1_W']) + p['dec1_b'], alpha=2.0)
    log = _mm(log1, p['dec2_W']) + p['dec2_b']
    return (out, log)


# speedup vs baseline: 5.2809x; 1.0042x over previous
import functools

import jax
import jax.numpy as jnp
from jax import lax
from jax.experimental import pallas as pl
from jax.experimental.pallas import tpu as pltpu
from jax.experimental.pallas import tpu_sc as plsc

N = 10000
E = 320000
D = 128
HEADS = 8
HD = 16

NC = 2
NS = 16
NW = NC * NS
CHUNK = 128
NPAD = 10240
EPT = 10240
EPAD = NW * EPT
NCHUNK = EPT // CHUNK
RPT_SC = NPAD // NS


def _mesh():
    return plsc.VectorSubcoreMesh(core_axis_name="c", subcore_axis_name="s",
                                  num_cores=NC, num_subcores=NS)


DW = D // NC
EPT2 = EPAD // NS
NCHUNK2 = EPT2 // CHUNK


@functools.cache
def _build_sc_conv():
    @functools.partial(
        pl.kernel,
        out_type=jax.ShapeDtypeStruct((NC, NPAD, DW), jnp.float32),
        mesh=_mesh(),
        compiler_params=pltpu.CompilerParams(use_tc_tiling_on_sc=False),
        scratch_types=[
            pltpu.VMEM((NCHUNK2, CHUNK), jnp.int32),
            pltpu.VMEM((NCHUNK2, CHUNK), jnp.int32),
            pltpu.VMEM((4, CHUNK, DW), jnp.float32),
            pltpu.VMEM_SHARED((NPAD, DW), jnp.float32),
            [pltpu.SemaphoreType.DMA] * 4,
            [pltpu.SemaphoreType.DMA] * 4,
        ],
    )
    def sc_conv(table_h, src_h, dst_h, zeros_h, out_h, sidx, didx, rows, acc, gsem, ssem):
        cid = lax.axis_index("c")
        sid = lax.axis_index("s")
        pltpu.sync_copy(zeros_h, acc.at[pl.ds(sid * RPT_SC, RPT_SC)])
        pltpu.sync_copy(src_h.at[pl.ds(sid * NCHUNK2, NCHUNK2)], sidx)
        pltpu.sync_copy(dst_h.at[pl.ds(sid * NCHUNK2, NCHUNK2)], didx)
        plsc.subcore_barrier()

        tab = table_h.at[cid]
        pltpu.async_copy(tab.at[sidx.at[0]], rows.at[0], gsem[0])
        pltpu.async_copy(tab.at[sidx.at[1]], rows.at[1], gsem[1])

        def body(t, carry):
            for b in range(4):
                k = 4 * t + b
                b2 = (b + 2) % 4
                pltpu.make_async_copy(tab.at[sidx.at[k]], rows.at[b], gsem[b]).wait()
                pltpu.async_copy(rows.at[b], acc.at[didx.at[k]], ssem[b], add=True)

                @pl.when(k >= 2)
                def _():
                    pltpu.make_async_copy(rows.at[b2], acc.at[didx.at[k]],
                                          ssem[b2]).wait()

                @pl.when(k + 2 < NCHUNK2)
                def _():
                    pltpu.async_copy(tab.at[sidx.at[k + 2]], rows.at[b2], gsem[b2])

            return carry

        lax.fori_loop(0, NCHUNK2 // 4, body, 0)
        pltpu.make_async_copy(rows.at[2], acc.at[didx.at[0]], ssem[2]).wait()
        pltpu.make_async_copy(rows.at[3], acc.at[didx.at[0]], ssem[3]).wait()
        plsc.subcore_barrier()
        pltpu.sync_copy(acc.at[pl.ds(sid * RPT_SC, RPT_SC)],
                        out_h.at[cid, pl.ds(sid * RPT_SC, RPT_SC)])

    return sc_conv


def _sc_conv(table, src2d, dst2d, zeros_sc):
    tsplit = table.reshape(NPAD, NC, DW).transpose(1, 0, 2)
    out = _build_sc_conv()(tsplit, src2d, dst2d, zeros_sc)
    return out.transpose(1, 0, 2).reshape(NPAD, D)


DEGW = 16


@functools.cache
def _build_sc_degree():
    @functools.partial(
        pl.kernel,
        out_type=jax.ShapeDtypeStruct((NC, NPAD, DEGW), jnp.float32),
        mesh=_mesh(),
        compiler_params=pltpu.CompilerParams(use_tc_tiling_on_sc=False),
        scratch_types=[
            pltpu.VMEM((NCHUNK, CHUNK), jnp.int32),
            pltpu.VMEM((CHUNK, DEGW), jnp.float32),
            pltpu.VMEM_SHARED((NPAD, DEGW), jnp.float32),
        ],
    )
    def sc_degree(dst_h, ones_h, zeros_h, out_h, didx, ones_v, acc):
        cid = lax.axis_index("c")
        sid = lax.axis_index("s")
        g = cid * NS + sid
        pltpu.sync_copy(dst_h.at[pl.ds(g * NCHUNK, NCHUNK)], didx)
        pltpu.sync_copy(ones_h, ones_v)
        pltpu.sync_copy(zeros_h, acc.at[pl.ds(sid * RPT_SC, RPT_SC)])
        plsc.subcore_barrier()

        def body(k, c):
            pltpu.sync_copy(ones_v, acc.at[didx.at[k]], add=True)
            return c

        lax.fori_loop(0, NCHUNK, body, 0)
        plsc.subcore_barrier()
        pltpu.sync_copy(acc.at[pl.ds(sid * RPT_SC, RPT_SC)],
                        out_h.at[cid, pl.ds(sid * RPT_SC, RPT_SC)])

    return sc_degree


def _sc_degree(dst2d):
    ones_rows = jnp.ones((CHUNK, DEGW), jnp.float32)
    zeros_rows = jnp.zeros((RPT_SC, DEGW), jnp.float32)
    return _build_sc_degree()(dst2d, ones_rows, zeros_rows)[:, :, 0]


@functools.cache
def _build_sc_take():
    @functools.partial(
        pl.kernel,
        out_type=jax.ShapeDtypeStruct((8192, D), jnp.float32),
        mesh=_mesh(),
        compiler_params=pltpu.CompilerParams(use_tc_tiling_on_sc=False),
        scratch_types=[
            pltpu.VMEM((2, CHUNK), jnp.int32),
            pltpu.VMEM((2 * CHUNK, D), jnp.float32),
            pltpu.SemaphoreType.DMA,
        ],
    )
    def sc_take(table_h, idx_h, out_h, gidx, rows, sem):
        cid = lax.axis_index("c")
        sid = lax.axis_index("s")
        g = cid * NS + sid
        pltpu.sync_copy(idx_h.at[pl.ds(g * 2, 2)], gidx)
        pltpu.async_copy(table_h.at[gidx.at[0]], rows.at[pl.ds(0, CHUNK)], sem).wait()
        pltpu.async_copy(table_h.at[gidx.at[1]], rows.at[pl.ds(CHUNK, CHUNK)], sem).wait()
        pltpu.sync_copy(rows, out_h.at[pl.ds(g * 2 * CHUNK, 2 * CHUNK)])

    return sc_take


def _sc_take(table, idx2d):
    return _build_sc_take()(table, idx2d)


def _pad_rows(a, m):
    r = a.shape[0] % m
    if r == 0:
        return a
    return jnp.concatenate([a, jnp.zeros((m - r,) + a.shape[1:], a.dtype)], axis=0)


def _mm(a, w, block_m=512):
    m, k = a.shape
    ko = w.shape[1]
    kop = ((ko + 127) // 128) * 128
    if kop != ko:
        w = jnp.concatenate([w, jnp.zeros((k, kop - ko), w.dtype)], axis=1)
    ap = _pad_rows(a, block_m)
    grid = ap.shape[0] // block_m

    def body(ar, wr, orr):
        orr[...] = jnp.dot(ar[...], wr[...], preferred_element_type=jnp.float32)

    out = pl.pallas_call(
        body,
        grid=(grid,),
        in_specs=[
            pl.BlockSpec((block_m, k), lambda i: (i, 0)),
            pl.BlockSpec((k, kop), lambda i: (0, 0)),
        ],
        out_specs=pl.BlockSpec((block_m, kop), lambda i: (i, 0)),
        out_shape=jax.ShapeDtypeStruct((ap.shape[0], kop), jnp.float32),
    )(ap, w)
    if kop != ko:
        out = out[:, :ko]
    if out.shape[0] != m:
        out = out[:m]
    return out


def _colstats(a, block_m=1024):
    ap = _pad_rows(a, block_m)
    grid = ap.shape[0] // block_m

    def body(ar, orr):
        @pl.when(pl.program_id(0) == 0)
        def _():
            orr[...] = jnp.zeros_like(orr)

        x = ar[...]
        s = jnp.sum(x, axis=0, keepdims=True)
        sq = jnp.sum(x * x, axis=0, keepdims=True)
        orr[...] += jnp.concatenate([s, sq, jnp.zeros((6, D), jnp.float32)], axis=0)

    return pl.pallas_call(
        body,
        grid=(grid,),
        in_specs=[pl.BlockSpec((block_m, D), lambda i: (i, 0))],
        out_specs=pl.BlockSpec((8, D), lambda i: (0, 0)),
        out_shape=jax.ShapeDtypeStruct((8, D), jnp.float32),
    )(ap)


def _celu2(x):
    return jnp.where(x > 0, x, 2.0 * (jnp.exp(0.5 * x) - 1.0))


def _dfam_kernel(qkv, epad, epadt, aggw, agg2v, pvec, block_m=512):
    grid = NPAD // block_m

    def body(q0r, q1r, q2r, k0r, k1r, k2r, v0r, v1r, v2r, er, etr, awr, a2r, pvr, orr):
        scale = float(HD) ** -0.5
        q = [q0r[...], q1r[...], q2r[...]]
        kk = [k0r[...], k1r[...], k2r[...]]
        v = [v0r[...], v1r[...], v2r[...]]
        e = er[...]
        et = etr[...]
        pv = pvr[...]
        a1b = pv[0:1, :]
        alpha = pv[1:2, :]
        dw = pv[2:3, :]
        db = pv[3:4, :]
        a2 = a2r[...]

        outs = []
        ws = []
        for i in range(3):
            s = [jnp.dot(q[i] * kk[j], e, preferred_element_type=jnp.float32) * scale
                 for j in range(3)]
            mx = jnp.maximum(jnp.maximum(s[0], s[1]), s[2])
            ex = [jnp.exp(sj - mx) for sj in s]
            z = ex[0] + ex[1] + ex[2]
            o_i = jnp.zeros_like(q[0])
            for j in range(3):
                p_l = jnp.dot(ex[j] / z, et, preferred_element_type=jnp.float32)
                o_i = o_i + p_l * v[j]
            o_i = _celu2(o_i)
            d_i = jnp.tanh(alpha * (jnp.dot(o_i, awr[...], preferred_element_type=jnp.float32) + a1b)) * dw + db
            w_i = jnp.sum(d_i * a2, axis=1, keepdims=True)
            outs.append(o_i)
            ws.append(w_i)
        wm = jnp.maximum(jnp.maximum(ws[0], ws[1]), ws[2])
        ew = [jnp.exp(wi - wm) for wi in ws]
        zw = ew[0] + ew[1] + ew[2]
        orr[...] = (ew[0] * outs[0] + ew[1] * outs[1] + ew[2] * outs[2]) / zw

    def bspec(c0):
        return pl.BlockSpec((block_m, D), lambda i: (i, c0))

    def cspec(r, c):
        return pl.BlockSpec((r, c), lambda i: (0, 0))

    q0, q1, q2 = qkv[0:NPAD], qkv[NPAD:2 * NPAD], qkv[2 * NPAD:3 * NPAD]

    return pl.pallas_call(
        body,
        grid=(grid,),
        in_specs=[bspec(0), bspec(0), bspec(0),
                  bspec(1), bspec(1), bspec(1),
                  bspec(2), bspec(2), bspec(2),
                  cspec(D, D), cspec(D, D), cspec(D, D),
                  cspec(1, D), cspec(8, D)],
        out_specs=pl.BlockSpec((block_m, D), lambda i: (i, 0)),
        out_shape=jax.ShapeDtypeStruct((NPAD, D), jnp.float32),
    )(q0, q1, q2, q0, q1, q2, q0, q1, q2, epad, epadt, aggw, agg2v, pvec)


def _dtf(x, p):
    return jnp.tanh(p['alpha'] * x) * p['weight'] + p['bias']


def _dgct_gate(colsq, p, eps=1e-05):
    emb = jnp.power(colsq + eps, 0.5) * p['alpha']
    gamma = _dtf(emb, p['dyt_gamma'])
    norm = gamma / jnp.power(jnp.mean(emb * emb, axis=1, keepdims=True) + eps, 0.5)
    beta = _dtf(emb, p['dyt_beta'])
    return 1.0 + jax.nn.celu(emb * norm + beta, alpha=2.0)


def _prep_edges(ei):
    pad = EPAD - E
    src = jnp.concatenate([ei[0], jnp.full((pad,), N, jnp.int32)]).reshape(EPAD // CHUNK, CHUNK)
    dst = jnp.concatenate([ei[1], jnp.full((pad,), N, jnp.int32)]).reshape(EPAD // CHUNK, CHUNK)
    return src, dst


def _conv(xw_pad, dis_pad, src2d, dst2d, b, zeros_sc):
    xwp = dis_pad * xw_pad
    s = _sc_conv(xwp, src2d, dst2d, zeros_sc)
    return dis_pad * (s + xwp) + b


def _encoder(x_pad, colsq, src2d, dst2d, dis_pad, p, zeros_sc):
    g1 = _dgct_gate(colsq, p['gct1'])
    g2 = _dgct_gate(colsq, p['gct2'])
    x0 = x_pad * g1
    x00 = x_pad * g2
    xw = _mm(jnp.concatenate([x0, x00], axis=0), p['gc1a_W'])
    ca1 = _conv(xw[:NPAD], dis_pad, src2d, dst2d, p['gc1a_b'], zeros_sc)
    cb1 = _conv(xw[NPAD:], dis_pad, src2d, dst2d, p['gc1a_b'], zeros_sc)
    x1a = _dtf(jax.nn.celu(ca1, alpha=2.0), p['norm1'])
    x1b = _dtf(cb1, p['norm1'])
    xw2 = _mm(jnp.concatenate([x1a, x1b], axis=0), p['gc2_W'])
    ca2 = _conv(xw2[:NPAD], dis_pad, src2d, dst2d, p['gc2_b'], zeros_sc)
    cb2 = _conv(xw2[NPAD:], dis_pad, src2d, dst2d, p['gc2_b'], zeros_sc)
    x1aa = _dtf(jax.nn.celu(ca2, alpha=2.0), p['norm2'])
    x1bb = jax.nn.sigmoid(cb2)
    return x1aa * x1bb


def kernel(x_s, edge_index_s, x_f, edge_index_f, idx, params):
    p = params
    zeros_sc = jnp.zeros((RPT_SC, DW), jnp.float32)

    src_s, dst_s = _prep_edges(edge_index_s)
    src_f, dst_f = _prep_edges(edge_index_f)

    degp_s = _sc_degree(dst_s)
    degp_f = _sc_degree(dst_f)
    deg_s = degp_s[0] + degp_s[1] + 1.0
    deg_f = degp_f[0] + degp_f[1] + 1.0
    dis_s = lax.rsqrt(deg_s)
    dis_f = lax.rsqrt(deg_f)
    rowmask = (jnp.arange(NPAD) < N).astype(jnp.float32)
    dis_s_pad = (dis_s * rowmask)[:, None]
    dis_f_pad = (dis_f * rowmask)[:, None]

    xs_pad = _pad_rows(x_s, NPAD)
    xf_pad = _pad_rows(x_f, NPAD)
    colsq_xs = _colstats(x_s)[1:2, :]
    colsq_xf = _colstats(x_f)[1:2, :]

    h1 = _encoder(xs_pad, colsq_xs, src_s, dst_s, dis_s_pad, p['enc1'], zeros_sc)
    h2 = _encoder(xs_pad, colsq_xs, src_f, dst_f, dis_f_pad, p['enc2'], zeros_sc)
    h3 = _encoder(xf_pad, colsq_xf, src_s, dst_s, dis_s_pad, p['enc1'], zeros_sc)
    h4 = _encoder(xf_pad, colsq_xf, src_f, dst_f, dis_f_pad, p['enc2'], zeros_sc)
    h5 = _encoder(xs_pad, colsq_xs, src_s, dst_s, dis_s_pad, p['enc3'], zeros_sc)
    h6 = _encoder(xs_pad, colsq_xs, src_f, dst_f, dis_f_pad, p['enc3'], zeros_sc)

    hloc = jax.nn.celu(
        _mm(jnp.concatenate([h1, h2, h3, h4], axis=0), p['local_W']) + p['local_b'],
        alpha=2.0)
    h1l, h2l, h3l, h4l = (hloc[i * NPAD:(i + 1) * NPAD] for i in range(4))
    h_com = (h5 + h6) * 0.5

    m1 = _colstats(h1l[:N])[0] / N
    m2 = _colstats(h2l[:N])[0] / N
    cc = jax.nn.sigmoid(_mm(jnp.stack([m1, m2], axis=0), p['global_W']) + p['global_b'])
    c1, c2 = cc[0], cc[1]

    wc = _mm(p['disc_W'], jnp.stack([c1, c2], axis=1))
    hcat = jnp.concatenate([h1l, h3l, h2l, h4l], axis=0)
    sc_all = _mm(hcat, wc)
    sc1 = sc_all[0:N, 0] + p['disc_b']
    sc3 = sc_all[NPAD:NPAD + N, 0] + p['disc_b']
    sc2 = sc_all[2 * NPAD:2 * NPAD + N, 1] + p['disc_b']
    sc4 = sc_all[3 * NPAD:3 * NPAD + N, 1] + p['disc_b']
    out = jax.nn.celu(jnp.concatenate([sc1, sc2, sc3, sc4]), alpha=2.0)

    ap = p['attn']
    wqkv = jnp.concatenate([ap['q_W'], ap['k_W'], ap['v_W']], axis=1)
    bqkv = jnp.concatenate([ap['q_b'], ap['k_b'], ap['v_b']])
    emb_rows = jnp.concatenate([h1l, h2l, h_com], axis=0)
    qkv = _mm(emb_rows, wqkv) + bqkv

    eye8 = jnp.eye(HEADS, dtype=jnp.float32)
    epad_small = jnp.repeat(eye8, HD, axis=0)
    epad = jnp.concatenate([epad_small, jnp.zeros((D, D - HEADS), jnp.float32)], axis=1)
    epadt = epad.T
    pvec = jnp.zeros((8, D), jnp.float32)
    pvec = pvec.at[0].set(ap['agg1_b'])
    pvec = pvec.at[1].set(jnp.full((D,), ap['agg_dtf']['alpha'][0]))
    pvec = pvec.at[2].set(ap['agg_dtf']['weight'])
    pvec = pvec.at[3].set(ap['agg_dtf']['bias'])
    agg2v = ap['agg2_W'].reshape(1, D)
    agg = _dfam_kernel(qkv, epad, epadt, ap['agg1_W'], agg2v, pvec)

    idx_cat = jnp.concatenate([idx[0], idx[1] + 386]).reshape(NW * 2, CHUNK)
    e12 = _sc_take(agg, idx_cat)
    e1, e2 = e12[:4096], e12[4096:]
    feature = jnp.concatenate([e1 + e2, e1 * e2, e1, e2], axis=1)
    log1 = jax.nn.celu(_mm(feature, p['dec1_W']) + p['dec1_b'], alpha=2.0)
    log = _mm(log1, p['dec2_W']) + p['dec2_b']
    return (out, log)
